# baseline (speedup 1.0000x reference)
import functools

import jax
import jax.numpy as jnp
from jax import lax
from jax.experimental import pallas as pl
from jax.experimental.pallas import tpu as pltpu
from jax.experimental.pallas import tpu_sc as plsc

NPTS = 50000
LANES = 128
ROWS = 392
NPAD = ROWS * LANES
NCLUST = 1024
KNN = 16
CIN = 128
COUT = 128
MROWS = 2000
NBLK = NPTS // MROWS


def _iota2(shape):
    r = lax.broadcasted_iota(jnp.int32, shape, 0)
    l = lax.broadcasted_iota(jnp.int32, shape, 1)
    return r * LANES + l


def _fps_body(px_ref, py_ref, pz_ref, idxs_ref, subpos_ref, p2_ref, d_ref):
    px = px_ref[...]
    py = py_ref[...]
    pz = pz_ref[...]
    flat = _iota2((ROWS, LANES))
    lane1 = lax.broadcasted_iota(jnp.int32, (1, LANES), 1)
    valid = flat < NPTS

    x0 = px[0:1, 0:1]
    y0 = py[0:1, 0:1]
    z0 = pz[0:1, 0:1]
    dx = px - x0
    dy = py - y0
    dz = pz - z0
    d = (dx * dx + dy * dy) + dz * dz
    d = jnp.where(valid, d, -1.0)
    d_ref[...] = d

    p2_ref[...] = (px * px + py * py) + pz * pz

    idxs_ref[0] = jnp.int32(0)
    subpos_ref[0] = x0[0, 0]
    subpos_ref[1] = y0[0, 0]
    subpos_ref[2] = z0[0, 0]

    def body(i, carry):
        d = d_ref[...]
        m = jnp.max(d)
        nxt = jnp.min(jnp.where(d == m, flat, jnp.int32(NPAD)))
        idxs_ref[i] = nxt
        r = nxt // LANES
        l = nxt % LANES
        lm = lane1 == l
        rx = px_ref[pl.ds(r, 1), :]
        ry = py_ref[pl.ds(r, 1), :]
        rz = pz_ref[pl.ds(r, 1), :]
        nx = jnp.sum(jnp.where(lm, rx, 0.0))
        ny = jnp.sum(jnp.where(lm, ry, 0.0))
        nz = jnp.sum(jnp.where(lm, rz, 0.0))
        subpos_ref[3 * i] = nx
        subpos_ref[3 * i + 1] = ny
        subpos_ref[3 * i + 2] = nz
        ddx = px - nx
        ddy = py - ny
        ddz = pz - nz
        dd = (ddx * ddx + ddy * ddy) + ddz * ddz
        d_ref[...] = jnp.minimum(d, dd)
        return carry

    lax.fori_loop(1, NCLUST, body, jnp.int32(0))


def _fps(px, py, pz, interpret=False):
    return pl.pallas_call(
        _fps_body,
        out_shape=[
            jax.ShapeDtypeStruct((NCLUST,), jnp.int32),
            jax.ShapeDtypeStruct((NCLUST * 3,), jnp.float32),
            jax.ShapeDtypeStruct((ROWS, LANES), jnp.float32),
        ],
        in_specs=[pl.BlockSpec(memory_space=pltpu.VMEM)] * 3,
        out_specs=[
            pl.BlockSpec(memory_space=pltpu.SMEM),
            pl.BlockSpec(memory_space=pltpu.SMEM),
            pl.BlockSpec(memory_space=pltpu.VMEM),
        ],
        scratch_shapes=[pltpu.VMEM((ROWS, LANES), jnp.float32)],
        interpret=interpret,
    )(px, py, pz)


CB = 4
NTIL = ROWS // 8
BIGI = 2 ** 30
INF = float("inf")


def _knn_body(px_ref, py_ref, pz_ref, p2_ref, subpos_ref, nbr_ref, flag_ref):
    g = pl.program_id(0)
    pos_idx = lax.broadcasted_iota(jnp.int32, (8, LANES), 0) * LANES \
        + lax.broadcasted_iota(jnp.int32, (8, LANES), 1)
    last_valid = (NTIL - 1) * 1024 + pos_idx < NPTS

    coef = []
    for k in range(CB):
        c = g * CB + k
        cx = subpos_ref[3 * c]
        cy = subpos_ref[3 * c + 1]
        cz = subpos_ref[3 * c + 2]
        c2 = (cx * cx + cy * cy) + cz * cz
        coef.append((cx.astype(jnp.bfloat16).astype(jnp.float32),
                     cy.astype(jnp.bfloat16).astype(jnp.float32),
                     cz.astype(jnp.bfloat16).astype(jnp.float32), c2))

    def tile_d(t, k):
        r = pl.ds(8 * t, 8)
        pxb = px_ref[r, :].astype(jnp.float32)
        pyb = py_ref[r, :].astype(jnp.float32)
        pzb = pz_ref[r, :].astype(jnp.float32)
        cxb, cyb, czb, c2 = coef[k]
        dot = (cxb * pxb + cyb * pyb) + czb * pzb
        d = (c2 + p2_ref[r, :]) - 2.0 * dot
        if t == NTIL - 1:
            d = jnp.where(last_valid, d, INF)
        return d

    for k in range(CB):
        c = g * CB + k
        s1 = jnp.full((8, LANES), INF)
        s2 = jnp.full((8, LANES), INF)
        f1 = jnp.zeros((8, LANES), jnp.int32)
        f2 = jnp.zeros((8, LANES), jnp.int32)
        for t in range(NTIL):
            d = tile_d(t, k)
            fi = t * 1024 + pos_idx
            lt1 = d < s1
            lt2 = d < s2
            s2 = jnp.where(lt1, s1, jnp.where(lt2, d, s2))
            f2 = jnp.where(lt1, f1, jnp.where(lt2, fi, f2))
            s1 = jnp.where(lt1, d, s1)
            f1 = jnp.where(lt1, fi, f1)
        ecnt = jnp.zeros((8, LANES), jnp.int32)
        v16 = None
        for j in range(KNN):
            m = jnp.min(s1)
            v16 = m
            f = jnp.min(jnp.where(s1 == m, f1, jnp.int32(BIGI)))
            nbr_ref[KNN * c + j] = f
            hit = f1 == f
            ecnt = ecnt + hit.astype(jnp.int32)
            s1 = jnp.where(hit, s2, s1)
            f1 = jnp.where(hit, f2, f1)
            s2 = jnp.where(hit, INF, s2)

        flag_ref[0] = jnp.int32(0)

        @pl.when(jnp.max(ecnt) >= 2)
        def _():
            cnt = jnp.zeros((8, LANES), jnp.int32)
            for t in range(NTIL):
                cnt = cnt + (tile_d(t, k) <= v16).astype(jnp.int32)
            bad = jnp.max(jnp.where(ecnt >= 2, cnt, 0)) >= 3
            flag_ref[0] = bad.astype(jnp.int32)

        @pl.when(flag_ref[0] == 1)
        def _():
            acc = [tile_d(t, k) for t in range(NTIL)]
            for j in range(KNN):
                m = acc[0]
                for t in range(1, NTIL):
                    m = jnp.minimum(m, acc[t])
                mv = jnp.min(m)
                idx = jnp.int32(BIGI)
                for t in range(NTIL):
                    idx = jnp.minimum(idx, jnp.min(jnp.where(
                        acc[t] == mv, t * 1024 + pos_idx, jnp.int32(BIGI))))
                nbr_ref[KNN * c + j] = idx
                for t in range(NTIL):
                    acc[t] = jnp.where(t * 1024 + pos_idx == idx, INF, acc[t])


def _knn(px, py, pz, p2, subpos, interpret=False):
    return pl.pallas_call(
        _knn_body,
        grid=(NCLUST // CB,),
        out_shape=jax.ShapeDtypeStruct((NCLUST * KNN,), jnp.int32),
        in_specs=[pl.BlockSpec(memory_space=pltpu.VMEM)] * 4
        + [pl.BlockSpec(memory_space=pltpu.SMEM)],
        out_specs=pl.BlockSpec(memory_space=pltpu.SMEM),
        scratch_shapes=[pltpu.SMEM((1,), jnp.int32)],
        interpret=interpret,
    )(px, py, pz, p2, subpos)


def _stats_body(x_ref, w_ref, b_ref, sums_ref):
    blk = pl.program_id(0)
    h = jnp.dot(x_ref[...], w_ref[...], preferred_element_type=jnp.float32)
    h = h + b_ref[...]
    s1 = jnp.sum(h, axis=0, keepdims=True)
    s2 = jnp.sum(h * h, axis=0, keepdims=True)
    upd = jnp.concatenate([s1, s2], axis=0)

    @pl.when(blk == 0)
    def _():
        sums_ref[...] = jnp.zeros_like(sums_ref)

    sums_ref[...] += upd


def _stats(x, W, b2, interpret=False):
    return pl.pallas_call(
        _stats_body,
        grid=(NBLK,),
        out_shape=jax.ShapeDtypeStruct((2, COUT), jnp.float32),
        in_specs=[
            pl.BlockSpec((MROWS, CIN), lambda b: (b, 0)),
            pl.BlockSpec(memory_space=pltpu.VMEM),
            pl.BlockSpec(memory_space=pltpu.VMEM),
        ],
        out_specs=pl.BlockSpec((2, COUT), lambda b: (0, 0)),
        interpret=interpret,
    )(x, W, b2)


def _mlp_body(x_ref, w_ref, b_ref, g_ref, beta_ref, sums_ref, h_ref):
    h = jnp.dot(x_ref[...], w_ref[...], preferred_element_type=jnp.float32)
    h = h + b_ref[...]
    n = jnp.float32(NPTS)
    mean = sums_ref[0:1, :] / n
    var = sums_ref[1:2, :] / n - mean * mean
    std = jnp.sqrt(var + 1e-5)
    h = (h - mean) / std * g_ref[...] + beta_ref[...]
    h_ref[...] = jnp.maximum(h, 0.0)


def _mlp(x, W, b2, g2, beta2, sums, interpret=False):
    return pl.pallas_call(
        _mlp_body,
        grid=(NBLK,),
        out_shape=jax.ShapeDtypeStruct((NPTS, COUT), jnp.float32),
        in_specs=[
            pl.BlockSpec((MROWS, CIN), lambda b: (b, 0)),
            pl.BlockSpec(memory_space=pltpu.VMEM),
            pl.BlockSpec(memory_space=pltpu.VMEM),
            pl.BlockSpec(memory_space=pltpu.VMEM),
            pl.BlockSpec(memory_space=pltpu.VMEM),
            pl.BlockSpec(memory_space=pltpu.VMEM),
        ],
        out_specs=pl.BlockSpec((MROWS, COUT), lambda b: (b, 0)),
        interpret=interpret,
    )(x, W, b2, g2, beta2, sums)


NWORK = 32
CPW = NCLUST // NWORK
RPW = CPW * KNN


def _gmax_body(nbr_hbm, h_hbm, idxs_hbm, batch_hbm, out_hbm, sb_hbm,
               idx_v, rows_v, out_v, ci_v, cb_v, sem):
    wid = lax.axis_index("s") * 2 + lax.axis_index("c")
    base = wid * CPW
    pltpu.sync_copy(nbr_hbm.at[pl.ds(base * KNN, RPW)], idx_v)
    pltpu.async_copy(h_hbm.at[idx_v], rows_v, sem).wait()

    def cl(ci, carry):
        for col in range(COUT // 16):
            acc = rows_v[ci * KNN, pl.ds(col * 16, 16)]
            for r in range(1, KNN):
                acc = jnp.maximum(acc, rows_v[ci * KNN + r, pl.ds(col * 16, 16)])
            out_v[ci, pl.ds(col * 16, 16)] = acc
        return carry

    lax.fori_loop(0, CPW, cl, jnp.int32(0))
    pltpu.sync_copy(out_v, out_hbm.at[pl.ds(base, CPW)])

    pltpu.sync_copy(idxs_hbm.at[pl.ds(base, CPW)], ci_v)
    pltpu.async_copy(batch_hbm.at[ci_v], cb_v, sem).wait()
    pltpu.sync_copy(cb_v, sb_hbm.at[pl.ds(base, CPW)])


def _gmax(nbr_flat, h, idxs, batch):
    mesh = plsc.VectorSubcoreMesh(core_axis_name="c", subcore_axis_name="s")
    fn = functools.partial(
        pl.kernel,
        mesh=mesh,
        out_type=[
            jax.ShapeDtypeStruct((NCLUST, COUT), jnp.float32),
            jax.ShapeDtypeStruct((NCLUST,), jnp.int32),
        ],
        scratch_types=[
            pltpu.VMEM((RPW,), jnp.int32),
            pltpu.VMEM((RPW, COUT), jnp.float32),
            pltpu.VMEM((CPW, COUT), jnp.float32),
            pltpu.VMEM((CPW,), jnp.int32),
            pltpu.VMEM((CPW,), jnp.int32),
            pltpu.SemaphoreType.DMA,
        ],
    )(_gmax_body)
    return fn(nbr_flat, h, idxs, batch)


def kernel(x, pos, batch, W, b, gamma, beta):
    posp = jnp.pad(pos, ((0, NPAD - NPTS), (0, 0)))
    px = posp[:, 0].reshape(ROWS, LANES)
    py = posp[:, 1].reshape(ROWS, LANES)
    pz = posp[:, 2].reshape(ROWS, LANES)

    idxs, subpos_flat, p2 = _fps(px, py, pz)
    subpos = subpos_flat.reshape(NCLUST, 3)
    nbr = _knn(px.astype(jnp.bfloat16), py.astype(jnp.bfloat16),
               pz.astype(jnp.bfloat16), p2, subpos_flat)

    b2 = b.reshape(1, COUT)
    g2 = gamma.reshape(1, COUT)
    beta2 = beta.reshape(1, COUT)
    sums = _stats(x, W, b2)
    h = _mlp(x, W, b2, g2, beta2, sums)

    out, sub_batch = _gmax(nbr, h, idxs, batch)
    return (out, subpos, sub_batch)


# speedup vs baseline: 2.1905x; 2.1905x over previous
import functools

import jax
import jax.numpy as jnp
from jax import lax
from jax.experimental import pallas as pl
from jax.experimental.pallas import tpu as pltpu
from jax.experimental.pallas import tpu_sc as plsc

NPTS = 50000
LANES = 128
ROWS = 392
NPAD = ROWS * LANES
NCLUST = 1024
KNN = 16
CIN = 128
COUT = 128
MROWS = 2000
NBLK = NPTS // MROWS


def _iota2(shape):
    r = lax.broadcasted_iota(jnp.int32, shape, 0)
    l = lax.broadcasted_iota(jnp.int32, shape, 1)
    return r * LANES + l


def _fps_body(px_ref, py_ref, pz_ref, idxs_ref, subpos_ref, p2_ref, d_ref):
    px = px_ref[...]
    py = py_ref[...]
    pz = pz_ref[...]
    flat = _iota2((ROWS, LANES))
    lane1 = lax.broadcasted_iota(jnp.int32, (1, LANES), 1)
    valid = flat < NPTS

    x0 = px[0:1, 0:1]
    y0 = py[0:1, 0:1]
    z0 = pz[0:1, 0:1]
    dx = px - x0
    dy = py - y0
    dz = pz - z0
    d = (dx * dx + dy * dy) + dz * dz
    d = jnp.where(valid, d, -1.0)
    d_ref[...] = d

    p2_ref[...] = (px * px + py * py) + pz * pz

    idxs_ref[0] = jnp.int32(0)
    subpos_ref[0] = x0[0, 0]
    subpos_ref[1] = y0[0, 0]
    subpos_ref[2] = z0[0, 0]

    def body(i, carry):
        d = d_ref[...]
        m = jnp.max(d)
        nxt = jnp.min(jnp.where(d == m, flat, jnp.int32(NPAD)))
        idxs_ref[i] = nxt
        r = nxt // LANES
        l = nxt % LANES
        lm = lane1 == l
        rx = px_ref[pl.ds(r, 1), :]
        ry = py_ref[pl.ds(r, 1), :]
        rz = pz_ref[pl.ds(r, 1), :]
        nx = jnp.sum(jnp.where(lm, rx, 0.0))
        ny = jnp.sum(jnp.where(lm, ry, 0.0))
        nz = jnp.sum(jnp.where(lm, rz, 0.0))
        subpos_ref[3 * i] = nx
        subpos_ref[3 * i + 1] = ny
        subpos_ref[3 * i + 2] = nz
        ddx = px - nx
        ddy = py - ny
        ddz = pz - nz
        dd = (ddx * ddx + ddy * ddy) + ddz * ddz
        d_ref[...] = jnp.minimum(d, dd)
        return carry

    lax.fori_loop(1, NCLUST, body, jnp.int32(0))


def _fps(px, py, pz, interpret=False):
    return pl.pallas_call(
        _fps_body,
        out_shape=[
            jax.ShapeDtypeStruct((NCLUST,), jnp.int32),
            jax.ShapeDtypeStruct((NCLUST * 3,), jnp.float32),
            jax.ShapeDtypeStruct((ROWS, LANES), jnp.float32),
        ],
        in_specs=[pl.BlockSpec(memory_space=pltpu.VMEM)] * 3,
        out_specs=[
            pl.BlockSpec(memory_space=pltpu.SMEM),
            pl.BlockSpec(memory_space=pltpu.SMEM),
            pl.BlockSpec(memory_space=pltpu.VMEM),
        ],
        scratch_shapes=[pltpu.VMEM((ROWS, LANES), jnp.float32)],
        interpret=interpret,
    )(px, py, pz)


CB = 4
NTIL = ROWS // 8
BIGI = 2 ** 30
INF = float("inf")


def _knn_body(px_ref, py_ref, pz_ref, p2_ref, subpos_ref, nbr_ref, flag_ref):
    g = pl.program_id(0)
    pos_idx = lax.broadcasted_iota(jnp.int32, (8, LANES), 0) * LANES \
        + lax.broadcasted_iota(jnp.int32, (8, LANES), 1)
    pos_f = pos_idx.astype(jnp.float32)
    last_valid = (NTIL - 1) * 1024 + pos_idx < NPTS

    coef = []
    for k in range(CB):
        c = g * CB + k
        cx = subpos_ref[3 * c]
        cy = subpos_ref[3 * c + 1]
        cz = subpos_ref[3 * c + 2]
        c2 = (cx * cx + cy * cy) + cz * cz
        coef.append((cx.astype(jnp.bfloat16).astype(jnp.float32),
                     cy.astype(jnp.bfloat16).astype(jnp.float32),
                     cz.astype(jnp.bfloat16).astype(jnp.float32), c2))

    def tile_d(t, k):
        r = pl.ds(8 * t, 8)
        pxb = px_ref[r, :].astype(jnp.float32)
        pyb = py_ref[r, :].astype(jnp.float32)
        pzb = pz_ref[r, :].astype(jnp.float32)
        cxb, cyb, czb, c2 = coef[k]
        dot = (cxb * pxb + cyb * pyb) + czb * pzb
        d = (c2 + p2_ref[r, :]) - 2.0 * dot
        if t == NTIL - 1:
            d = jnp.where(last_valid, d, INF)
        return d

    BIGF = jnp.float32(3e38)
    s1 = [None] * CB
    s2 = [None] * CB
    f1 = [None] * CB
    f2 = [None] * CB
    ecnt = [None] * CB
    v16 = [None] * CB
    for k in range(CB):
        s1[k] = jnp.full((8, LANES), INF)
        s2[k] = jnp.full((8, LANES), INF)
        f1[k] = jnp.zeros((8, LANES), jnp.float32)
        f2[k] = jnp.zeros((8, LANES), jnp.float32)
        ecnt[k] = jnp.zeros((8, LANES), jnp.float32)
    for t in range(NTIL):
        fi = float(t * 1024) + pos_f
        for k in range(CB):
            d = tile_d(t, k)
            lt1 = d < s1[k]
            lt2 = d < s2[k]
            s2[k] = jnp.where(lt1, s1[k], jnp.where(lt2, d, s2[k]))
            f2[k] = jnp.where(lt1, f1[k], jnp.where(lt2, fi, f2[k]))
            s1[k] = jnp.where(lt1, d, s1[k])
            f1[k] = jnp.where(lt1, fi, f1[k])
    for j in range(KNN):
        for k in range(CB):
            m = jnp.min(s1[k])
            v16[k] = m
            f = jnp.min(jnp.where(s1[k] == m, f1[k], BIGF))
            nbr_ref[KNN * (g * CB + k) + j] = f.astype(jnp.int32)
            hit = f1[k] == f
            ecnt[k] = ecnt[k] + hit.astype(jnp.float32)
            s1[k] = jnp.where(hit, s2[k], s1[k])
            f1[k] = jnp.where(hit, f2[k], f1[k])
            s2[k] = jnp.where(hit, INF, s2[k])

    for k in range(CB):
        c = g * CB + k
        flag_ref[0] = jnp.int32(0)

        @pl.when(jnp.max(ecnt[k]) >= 2.0)
        def _():
            cnt = jnp.zeros((8, LANES), jnp.float32)
            for t in range(NTIL):
                cnt = cnt + (tile_d(t, k) <= v16[k]).astype(jnp.float32)
            bad = jnp.max(jnp.where(ecnt[k] >= 2.0, cnt, 0.0)) >= 3.0
            flag_ref[0] = bad.astype(jnp.int32)

        @pl.when(flag_ref[0] == 1)
        def _():
            acc = [tile_d(t, k) for t in range(NTIL)]
            for j in range(KNN):
                m = acc[0]
                for t in range(1, NTIL):
                    m = jnp.minimum(m, acc[t])
                mv = jnp.min(m)
                idx = jnp.float32(BIGI)
                for t in range(NTIL):
                    idx = jnp.minimum(idx, jnp.min(jnp.where(
                        acc[t] == mv, float(t * 1024) + pos_f, BIGF)))
                nbr_ref[KNN * c + j] = idx.astype(jnp.int32)
                for t in range(NTIL):
                    acc[t] = jnp.where(float(t * 1024) + pos_f == idx,
                                       INF, acc[t])


def _knn(px, py, pz, p2, subpos, interpret=False):
    return pl.pallas_call(
        _knn_body,
        grid=(NCLUST // CB,),
        out_shape=jax.ShapeDtypeStruct((NCLUST * KNN,), jnp.int32),
        in_specs=[pl.BlockSpec(memory_space=pltpu.VMEM)] * 4
        + [pl.BlockSpec(memory_space=pltpu.SMEM)],
        out_specs=pl.BlockSpec(memory_space=pltpu.SMEM),
        scratch_shapes=[pltpu.SMEM((1,), jnp.int32)],
        interpret=interpret,
    )(px, py, pz, p2, subpos)


def _stats_body(x_ref, w_ref, b_ref, sums_ref):
    blk = pl.program_id(0)
    h = jnp.dot(x_ref[...], w_ref[...], preferred_element_type=jnp.float32)
    h = h + b_ref[...]
    s1 = jnp.sum(h, axis=0, keepdims=True)
    s2 = jnp.sum(h * h, axis=0, keepdims=True)
    upd = jnp.concatenate([s1, s2], axis=0)

    @pl.when(blk == 0)
    def _():
        sums_ref[...] = jnp.zeros_like(sums_ref)

    sums_ref[...] += upd


def _stats(x, W, b2, interpret=False):
    return pl.pallas_call(
        _stats_body,
        grid=(NBLK,),
        out_shape=jax.ShapeDtypeStruct((2, COUT), jnp.float32),
        in_specs=[
            pl.BlockSpec((MROWS, CIN), lambda b: (b, 0)),
            pl.BlockSpec(memory_space=pltpu.VMEM),
            pl.BlockSpec(memory_space=pltpu.VMEM),
        ],
        out_specs=pl.BlockSpec((2, COUT), lambda b: (0, 0)),
        interpret=interpret,
    )(x, W, b2)


def _mlp_body(x_ref, w_ref, b_ref, g_ref, beta_ref, sums_ref, h_ref):
    h = jnp.dot(x_ref[...], w_ref[...], preferred_element_type=jnp.float32)
    h = h + b_ref[...]
    n = jnp.float32(NPTS)
    mean = sums_ref[0:1, :] / n
    var = sums_ref[1:2, :] / n - mean * mean
    std = jnp.sqrt(var + 1e-5)
    h = (h - mean) / std * g_ref[...] + beta_ref[...]
    h_ref[...] = jnp.maximum(h, 0.0)


def _mlp(x, W, b2, g2, beta2, sums, interpret=False):
    return pl.pallas_call(
        _mlp_body,
        grid=(NBLK,),
        out_shape=jax.ShapeDtypeStruct((NPTS, COUT), jnp.float32),
        in_specs=[
            pl.BlockSpec((MROWS, CIN), lambda b: (b, 0)),
            pl.BlockSpec(memory_space=pltpu.VMEM),
            pl.BlockSpec(memory_space=pltpu.VMEM),
            pl.BlockSpec(memory_space=pltpu.VMEM),
            pl.BlockSpec(memory_space=pltpu.VMEM),
            pl.BlockSpec(memory_space=pltpu.VMEM),
        ],
        out_specs=pl.BlockSpec((MROWS, COUT), lambda b: (b, 0)),
        interpret=interpret,
    )(x, W, b2, g2, beta2, sums)


NWORK = 32
CPW = NCLUST // NWORK
RPW = CPW * KNN


def _gmax_body(nbr_hbm, h_hbm, idxs_hbm, batch_hbm, out_hbm, sb_hbm,
               idx_v, rows_v, out_v, ci_v, cb_v, sem):
    wid = lax.axis_index("s") * 2 + lax.axis_index("c")
    base = wid * CPW
    pltpu.sync_copy(nbr_hbm.at[pl.ds(base * KNN, RPW)], idx_v)
    pltpu.async_copy(h_hbm.at[idx_v], rows_v, sem).wait()

    def cl(ci, carry):
        for col in range(COUT // 16):
            acc = rows_v[ci * KNN, pl.ds(col * 16, 16)]
            for r in range(1, KNN):
                acc = jnp.maximum(acc, rows_v[ci * KNN + r, pl.ds(col * 16, 16)])
            out_v[ci, pl.ds(col * 16, 16)] = acc
        return carry

    lax.fori_loop(0, CPW, cl, jnp.int32(0))
    pltpu.sync_copy(out_v, out_hbm.at[pl.ds(base, CPW)])

    pltpu.sync_copy(idxs_hbm.at[pl.ds(base, CPW)], ci_v)
    pltpu.async_copy(batch_hbm.at[ci_v], cb_v, sem).wait()
    pltpu.sync_copy(cb_v, sb_hbm.at[pl.ds(base, CPW)])


def _gmax(nbr_flat, h, idxs, batch):
    mesh = plsc.VectorSubcoreMesh(core_axis_name="c", subcore_axis_name="s")
    fn = functools.partial(
        pl.kernel,
        mesh=mesh,
        out_type=[
            jax.ShapeDtypeStruct((NCLUST, COUT), jnp.float32),
            jax.ShapeDtypeStruct((NCLUST,), jnp.int32),
        ],
        scratch_types=[
            pltpu.VMEM((RPW,), jnp.int32),
            pltpu.VMEM((RPW, COUT), jnp.float32),
            pltpu.VMEM((CPW, COUT), jnp.float32),
            pltpu.VMEM((CPW,), jnp.int32),
            pltpu.VMEM((CPW,), jnp.int32),
            pltpu.SemaphoreType.DMA,
        ],
    )(_gmax_body)
    return fn(nbr_flat, h, idxs, batch)


def kernel(x, pos, batch, W, b, gamma, beta):
    posp = jnp.pad(pos, ((0, NPAD - NPTS), (0, 0)))
    px = posp[:, 0].reshape(ROWS, LANES)
    py = posp[:, 1].reshape(ROWS, LANES)
    pz = posp[:, 2].reshape(ROWS, LANES)

    idxs, subpos_flat, p2 = _fps(px, py, pz)
    subpos = subpos_flat.reshape(NCLUST, 3)
    nbr = _knn(px.astype(jnp.bfloat16), py.astype(jnp.bfloat16),
               pz.astype(jnp.bfloat16), p2, subpos_flat)

    b2 = b.reshape(1, COUT)
    g2 = gamma.reshape(1, COUT)
    beta2 = beta.reshape(1, COUT)
    sums = _stats(x, W, b2)
    h = _mlp(x, W, b2, g2, beta2, sums)

    out, sub_batch = _gmax(nbr, h, idxs, batch)
    return (out, subpos, sub_batch)


# speedup vs baseline: 2.1919x; 1.0006x over previous
import functools

import jax
import jax.numpy as jnp
from jax import lax
from jax.experimental import pallas as pl
from jax.experimental.pallas import tpu as pltpu
from jax.experimental.pallas import tpu_sc as plsc

NPTS = 50000
LANES = 128
ROWS = 392
NPAD = ROWS * LANES
NCLUST = 1024
KNN = 16
CIN = 128
COUT = 128
MROWS = 2000
NBLK = NPTS // MROWS


def _iota2(shape):
    r = lax.broadcasted_iota(jnp.int32, shape, 0)
    l = lax.broadcasted_iota(jnp.int32, shape, 1)
    return r * LANES + l


def _fps_body(px_ref, py_ref, pz_ref, idxs_ref, subpos_ref, p2_ref, d_ref):
    px = px_ref[...]
    py = py_ref[...]
    pz = pz_ref[...]
    flat = _iota2((ROWS, LANES))
    lane1 = lax.broadcasted_iota(jnp.int32, (1, LANES), 1)
    valid = flat < NPTS

    x0 = px[0:1, 0:1]
    y0 = py[0:1, 0:1]
    z0 = pz[0:1, 0:1]
    dx = px - x0
    dy = py - y0
    dz = pz - z0
    d = (dx * dx + dy * dy) + dz * dz
    d = jnp.where(valid, d, -1.0)
    d_ref[...] = d

    p2_ref[...] = (px * px + py * py) + pz * pz

    idxs_ref[0] = jnp.int32(0)
    subpos_ref[0] = x0[0, 0]
    subpos_ref[1] = y0[0, 0]
    subpos_ref[2] = z0[0, 0]

    def body(i, carry):
        d = d_ref[...]
        m = jnp.max(d)
        nxt = jnp.min(jnp.where(d == m, flat, jnp.int32(NPAD)))
        idxs_ref[i] = nxt
        r = nxt // LANES
        l = nxt % LANES
        lm = lane1 == l
        rx = px_ref[pl.ds(r, 1), :]
        ry = py_ref[pl.ds(r, 1), :]
        rz = pz_ref[pl.ds(r, 1), :]
        nx = jnp.sum(jnp.where(lm, rx, 0.0))
        ny = jnp.sum(jnp.where(lm, ry, 0.0))
        nz = jnp.sum(jnp.where(lm, rz, 0.0))
        subpos_ref[3 * i] = nx
        subpos_ref[3 * i + 1] = ny
        subpos_ref[3 * i + 2] = nz
        ddx = px - nx
        ddy = py - ny
        ddz = pz - nz
        dd = (ddx * ddx + ddy * ddy) + ddz * ddz
        d_ref[...] = jnp.minimum(d, dd)
        return carry

    lax.fori_loop(1, NCLUST, body, jnp.int32(0))


def _fps(px, py, pz, interpret=False):
    return pl.pallas_call(
        _fps_body,
        out_shape=[
            jax.ShapeDtypeStruct((NCLUST,), jnp.int32),
            jax.ShapeDtypeStruct((NCLUST * 3,), jnp.float32),
            jax.ShapeDtypeStruct((ROWS, LANES), jnp.float32),
        ],
        in_specs=[pl.BlockSpec(memory_space=pltpu.VMEM)] * 3,
        out_specs=[
            pl.BlockSpec(memory_space=pltpu.SMEM),
            pl.BlockSpec(memory_space=pltpu.SMEM),
            pl.BlockSpec(memory_space=pltpu.VMEM),
        ],
        scratch_shapes=[pltpu.VMEM((ROWS, LANES), jnp.float32)],
        interpret=interpret,
    )(px, py, pz)


CB = 8
NTIL = ROWS // 8
BIGI = 2 ** 30
INF = float("inf")


def _knn_body(px_ref, py_ref, pz_ref, p2_ref, subpos_ref, nbr_ref, flag_ref):
    g = pl.program_id(0)
    pos_idx = lax.broadcasted_iota(jnp.int32, (8, LANES), 0) * LANES \
        + lax.broadcasted_iota(jnp.int32, (8, LANES), 1)
    pos_f = pos_idx.astype(jnp.float32)
    last_valid = (NTIL - 1) * 1024 + pos_idx < NPTS

    coef = []
    for k in range(CB):
        c = g * CB + k
        cx = subpos_ref[3 * c]
        cy = subpos_ref[3 * c + 1]
        cz = subpos_ref[3 * c + 2]
        c2 = (cx * cx + cy * cy) + cz * cz
        coef.append((cx.astype(jnp.bfloat16).astype(jnp.float32),
                     cy.astype(jnp.bfloat16).astype(jnp.float32),
                     cz.astype(jnp.bfloat16).astype(jnp.float32), c2))

    def tile_d(t, k):
        r = pl.ds(8 * t, 8)
        pxb = px_ref[r, :].astype(jnp.float32)
        pyb = py_ref[r, :].astype(jnp.float32)
        pzb = pz_ref[r, :].astype(jnp.float32)
        cxb, cyb, czb, c2 = coef[k]
        dot = (cxb * pxb + cyb * pyb) + czb * pzb
        d = (c2 + p2_ref[r, :]) - 2.0 * dot
        if t == NTIL - 1:
            d = jnp.where(last_valid, d, INF)
        return d

    BIGF = jnp.float32(3e38)
    s1 = [None] * CB
    s2 = [None] * CB
    f1 = [None] * CB
    f2 = [None] * CB
    ecnt = [None] * CB
    v16 = [None] * CB
    for k in range(CB):
        s1[k] = jnp.full((8, LANES), INF)
        s2[k] = jnp.full((8, LANES), INF)
        f1[k] = jnp.zeros((8, LANES), jnp.float32)
        f2[k] = jnp.zeros((8, LANES), jnp.float32)
        ecnt[k] = jnp.zeros((8, LANES), jnp.float32)
    for t in range(NTIL):
        fi = float(t * 1024) + pos_f
        for k in range(CB):
            d = tile_d(t, k)
            lt1 = d < s1[k]
            lt2 = d < s2[k]
            s2[k] = jnp.where(lt1, s1[k], jnp.where(lt2, d, s2[k]))
            f2[k] = jnp.where(lt1, f1[k], jnp.where(lt2, fi, f2[k]))
            s1[k] = jnp.where(lt1, d, s1[k])
            f1[k] = jnp.where(lt1, fi, f1[k])
    for j in range(KNN):
        for k in range(CB):
            m = jnp.min(s1[k])
            v16[k] = m
            f = jnp.min(jnp.where(s1[k] == m, f1[k], BIGF))
            nbr_ref[KNN * (g * CB + k) + j] = f.astype(jnp.int32)
            hit = f1[k] == f
            ecnt[k] = ecnt[k] + hit.astype(jnp.float32)
            s1[k] = jnp.where(hit, s2[k], s1[k])
            f1[k] = jnp.where(hit, f2[k], f1[k])
            s2[k] = jnp.where(hit, INF, s2[k])

    for k in range(CB):
        c = g * CB + k
        flag_ref[0] = jnp.int32(0)

        @pl.when(jnp.max(ecnt[k]) >= 2.0)
        def _():
            cnt = jnp.zeros((8, LANES), jnp.float32)
            for t in range(NTIL):
                cnt = cnt + (tile_d(t, k) <= v16[k]).astype(jnp.float32)
            bad = jnp.max(jnp.where(ecnt[k] >= 2.0, cnt, 0.0)) >= 3.0
            flag_ref[0] = bad.astype(jnp.int32)

        @pl.when(flag_ref[0] == 1)
        def _():
            acc = [tile_d(t, k) for t in range(NTIL)]
            for j in range(KNN):
                m = acc[0]
                for t in range(1, NTIL):
                    m = jnp.minimum(m, acc[t])
                mv = jnp.min(m)
                idx = jnp.float32(BIGI)
                for t in range(NTIL):
                    idx = jnp.minimum(idx, jnp.min(jnp.where(
                        acc[t] == mv, float(t * 1024) + pos_f, BIGF)))
                nbr_ref[KNN * c + j] = idx.astype(jnp.int32)
                for t in range(NTIL):
                    acc[t] = jnp.where(float(t * 1024) + pos_f == idx,
                                       INF, acc[t])


def _knn(px, py, pz, p2, subpos, interpret=False):
    return pl.pallas_call(
        _knn_body,
        grid=(NCLUST // CB,),
        out_shape=jax.ShapeDtypeStruct((NCLUST * KNN,), jnp.int32),
        in_specs=[pl.BlockSpec(memory_space=pltpu.VMEM)] * 4
        + [pl.BlockSpec(memory_space=pltpu.SMEM)],
        out_specs=pl.BlockSpec(memory_space=pltpu.SMEM),
        scratch_shapes=[pltpu.SMEM((1,), jnp.int32)],
        interpret=interpret,
    )(px, py, pz, p2, subpos)


def _stats_body(x_ref, w_ref, b_ref, sums_ref):
    blk = pl.program_id(0)
    h = jnp.dot(x_ref[...], w_ref[...], preferred_element_type=jnp.float32)
    h = h + b_ref[...]
    s1 = jnp.sum(h, axis=0, keepdims=True)
    s2 = jnp.sum(h * h, axis=0, keepdims=True)
    upd = jnp.concatenate([s1, s2], axis=0)

    @pl.when(blk == 0)
    def _():
        sums_ref[...] = jnp.zeros_like(sums_ref)

    sums_ref[...] += upd


def _stats(x, W, b2, interpret=False):
    return pl.pallas_call(
        _stats_body,
        grid=(NBLK,),
        out_shape=jax.ShapeDtypeStruct((2, COUT), jnp.float32),
        in_specs=[
            pl.BlockSpec((MROWS, CIN), lambda b: (b, 0)),
            pl.BlockSpec(memory_space=pltpu.VMEM),
            pl.BlockSpec(memory_space=pltpu.VMEM),
        ],
        out_specs=pl.BlockSpec((2, COUT), lambda b: (0, 0)),
        interpret=interpret,
    )(x, W, b2)


def _mlp_body(x_ref, w_ref, b_ref, g_ref, beta_ref, sums_ref, h_ref):
    h = jnp.dot(x_ref[...], w_ref[...], preferred_element_type=jnp.float32)
    h = h + b_ref[...]
    n = jnp.float32(NPTS)
    mean = sums_ref[0:1, :] / n
    var = sums_ref[1:2, :] / n - mean * mean
    std = jnp.sqrt(var + 1e-5)
    h = (h - mean) / std * g_ref[...] + beta_ref[...]
    h_ref[...] = jnp.maximum(h, 0.0)


def _mlp(x, W, b2, g2, beta2, sums, interpret=False):
    return pl.pallas_call(
        _mlp_body,
        grid=(NBLK,),
        out_shape=jax.ShapeDtypeStruct((NPTS, COUT), jnp.float32),
        in_specs=[
            pl.BlockSpec((MROWS, CIN), lambda b: (b, 0)),
            pl.BlockSpec(memory_space=pltpu.VMEM),
            pl.BlockSpec(memory_space=pltpu.VMEM),
            pl.BlockSpec(memory_space=pltpu.VMEM),
            pl.BlockSpec(memory_space=pltpu.VMEM),
            pl.BlockSpec(memory_space=pltpu.VMEM),
        ],
        out_specs=pl.BlockSpec((MROWS, COUT), lambda b: (b, 0)),
        interpret=interpret,
    )(x, W, b2, g2, beta2, sums)


NWORK = 32
CPW = NCLUST // NWORK
RPW = CPW * KNN


def _gmax_body(nbr_hbm, h_hbm, idxs_hbm, batch_hbm, out_hbm, sb_hbm,
               idx_v, rows_v, out_v, ci_v, cb_v, sem):
    wid = lax.axis_index("s") * 2 + lax.axis_index("c")
    base = wid * CPW
    pltpu.sync_copy(nbr_hbm.at[pl.ds(base * KNN, RPW)], idx_v)
    pltpu.async_copy(h_hbm.at[idx_v], rows_v, sem).wait()

    def cl(ci, carry):
        for col in range(COUT // 16):
            acc = rows_v[ci * KNN, pl.ds(col * 16, 16)]
            for r in range(1, KNN):
                acc = jnp.maximum(acc, rows_v[ci * KNN + r, pl.ds(col * 16, 16)])
            out_v[ci, pl.ds(col * 16, 16)] = acc
        return carry

    lax.fori_loop(0, CPW, cl, jnp.int32(0))
    pltpu.sync_copy(out_v, out_hbm.at[pl.ds(base, CPW)])

    pltpu.sync_copy(idxs_hbm.at[pl.ds(base, CPW)], ci_v)
    pltpu.async_copy(batch_hbm.at[ci_v], cb_v, sem).wait()
    pltpu.sync_copy(cb_v, sb_hbm.at[pl.ds(base, CPW)])


def _gmax(nbr_flat, h, idxs, batch):
    mesh = plsc.VectorSubcoreMesh(core_axis_name="c", subcore_axis_name="s")
    fn = functools.partial(
        pl.kernel,
        mesh=mesh,
        out_type=[
            jax.ShapeDtypeStruct((NCLUST, COUT), jnp.float32),
            jax.ShapeDtypeStruct((NCLUST,), jnp.int32),
        ],
        scratch_types=[
            pltpu.VMEM((RPW,), jnp.int32),
            pltpu.VMEM((RPW, COUT), jnp.float32),
            pltpu.VMEM((CPW, COUT), jnp.float32),
            pltpu.VMEM((CPW,), jnp.int32),
            pltpu.VMEM((CPW,), jnp.int32),
            pltpu.SemaphoreType.DMA,
        ],
    )(_gmax_body)
    return fn(nbr_flat, h, idxs, batch)


def kernel(x, pos, batch, W, b, gamma, beta):
    posp = jnp.pad(pos, ((0, NPAD - NPTS), (0, 0)))
    px = posp[:, 0].reshape(ROWS, LANES)
    py = posp[:, 1].reshape(ROWS, LANES)
    pz = posp[:, 2].reshape(ROWS, LANES)

    idxs, subpos_flat, p2 = _fps(px, py, pz)
    subpos = subpos_flat.reshape(NCLUST, 3)
    nbr = _knn(px.astype(jnp.bfloat16), py.astype(jnp.bfloat16),
               pz.astype(jnp.bfloat16), p2, subpos_flat)

    b2 = b.reshape(1, COUT)
    g2 = gamma.reshape(1, COUT)
    beta2 = beta.reshape(1, COUT)
    sums = _stats(x, W, b2)
    h = _mlp(x, W, b2, g2, beta2, sums)

    out, sub_batch = _gmax(nbr, h, idxs, batch)
    return (out, subpos, sub_batch)


# speedup vs baseline: 5.2603x; 2.3998x over previous
import functools

import jax
import jax.numpy as jnp
from jax import lax
from jax.experimental import pallas as pl
from jax.experimental.pallas import tpu as pltpu
from jax.experimental.pallas import tpu_sc as plsc

NPTS = 50000
LANES = 128
ROWS = 392
NPAD = ROWS * LANES
NCLUST = 1024
KNN = 16
CIN = 128
COUT = 128
MROWS = 2000
NBLK = NPTS // MROWS


def _iota2(shape):
    r = lax.broadcasted_iota(jnp.int32, shape, 0)
    l = lax.broadcasted_iota(jnp.int32, shape, 1)
    return r * LANES + l


def _fps_body(px_ref, py_ref, pz_ref, idxs_ref, subpos_ref, p2_ref, d_ref):
    px = px_ref[...]
    py = py_ref[...]
    pz = pz_ref[...]
    flat = _iota2((ROWS, LANES))
    lane1 = lax.broadcasted_iota(jnp.int32, (1, LANES), 1)
    valid = flat < NPTS

    x0 = px[0:1, 0:1]
    y0 = py[0:1, 0:1]
    z0 = pz[0:1, 0:1]
    dx = px - x0
    dy = py - y0
    dz = pz - z0
    d = (dx * dx + dy * dy) + dz * dz
    d = jnp.where(valid, d, -1.0)
    d_ref[...] = d

    p2_ref[...] = (px * px + py * py) + pz * pz

    idxs_ref[0] = jnp.int32(0)
    subpos_ref[0] = x0[0, 0]
    subpos_ref[1] = y0[0, 0]
    subpos_ref[2] = z0[0, 0]

    def body(i, carry):
        d = d_ref[...]
        m = jnp.max(d)
        nxt = jnp.min(jnp.where(d == m, flat, jnp.int32(NPAD)))
        idxs_ref[i] = nxt
        r = nxt // LANES
        l = nxt % LANES
        lm = lane1 == l
        rx = px_ref[pl.ds(r, 1), :]
        ry = py_ref[pl.ds(r, 1), :]
        rz = pz_ref[pl.ds(r, 1), :]
        nx = jnp.sum(jnp.where(lm, rx, 0.0))
        ny = jnp.sum(jnp.where(lm, ry, 0.0))
        nz = jnp.sum(jnp.where(lm, rz, 0.0))
        subpos_ref[3 * i] = nx
        subpos_ref[3 * i + 1] = ny
        subpos_ref[3 * i + 2] = nz
        ddx = px - nx
        ddy = py - ny
        ddz = pz - nz
        dd = (ddx * ddx + ddy * ddy) + ddz * ddz
        d_ref[...] = jnp.minimum(d, dd)
        return carry

    lax.fori_loop(1, NCLUST, body, jnp.int32(0))


def _fps(px, py, pz, interpret=False):
    return pl.pallas_call(
        _fps_body,
        out_shape=[
            jax.ShapeDtypeStruct((NCLUST,), jnp.int32),
            jax.ShapeDtypeStruct((NCLUST * 3,), jnp.float32),
            jax.ShapeDtypeStruct((ROWS, LANES), jnp.float32),
        ],
        in_specs=[pl.BlockSpec(memory_space=pltpu.VMEM)] * 3,
        out_specs=[
            pl.BlockSpec(memory_space=pltpu.SMEM),
            pl.BlockSpec(memory_space=pltpu.SMEM),
            pl.BlockSpec(memory_space=pltpu.VMEM),
        ],
        scratch_shapes=[pltpu.VMEM((ROWS, LANES), jnp.float32)],
        interpret=interpret,
    )(px, py, pz)


CB = 8
NTIL = ROWS // 8
BIGI = 2 ** 30
INF = float("inf")


def _knn_body(px_ref, py_ref, pz_ref, p2_ref, subpos_ref, nbr_ref):
    g = pl.program_id(0)
    sub_i = lax.broadcasted_iota(jnp.int32, (8, LANES), 0)
    lane_i = lax.broadcasted_iota(jnp.int32, (8, LANES), 1)
    pos_idx = sub_i * LANES + lane_i
    pos_f = pos_idx.astype(jnp.float32)
    last_valid = (NTIL - 1) * 1024 + pos_idx < NPTS

    coef = []
    for k in range(CB):
        c = g * CB + k
        cx = subpos_ref[3 * c]
        cy = subpos_ref[3 * c + 1]
        cz = subpos_ref[3 * c + 2]
        c2 = (cx * cx + cy * cy) + cz * cz
        coef.append((cx.astype(jnp.bfloat16).astype(jnp.float32),
                     cy.astype(jnp.bfloat16).astype(jnp.float32),
                     cz.astype(jnp.bfloat16).astype(jnp.float32), c2))

    def tile_d(t, k):
        r = pl.ds(8 * t, 8)
        pxb = px_ref[r, :].astype(jnp.float32)
        pyb = py_ref[r, :].astype(jnp.float32)
        pzb = pz_ref[r, :].astype(jnp.float32)
        cxb, cyb, czb, c2 = coef[k]
        dot = (cxb * pxb + cyb * pyb) + czb * pzb
        d = (c2 + p2_ref[r, :]) - 2.0 * dot
        if t == NTIL - 1:
            d = jnp.where(last_valid, d, INF)
        return d

    BIGF = jnp.float32(3e38)

    T = [jnp.full((8, LANES), INF) for _ in range(3)]
    G = [jnp.zeros((8, LANES), jnp.float32) for _ in range(3)]
    OV = jnp.full((8, LANES), INF)

    for k in range(CB):
        s1 = jnp.full((8, LANES), INF)
        s2 = jnp.full((8, LANES), INF)
        f1 = jnp.zeros((8, LANES), jnp.float32)
        f2 = jnp.zeros((8, LANES), jnp.float32)
        ovp = jnp.full((8, LANES), INF)
        for t in range(NTIL):
            fi = float(t * 1024) + pos_f
            d = tile_d(t, k)
            lt1 = d < s1
            lt2 = d < s2
            ovp = jnp.minimum(ovp, jnp.where(lt2, s2, d))
            s2 = jnp.where(lt1, s1, jnp.where(lt2, d, s2))
            f2 = jnp.where(lt1, f1, jnp.where(lt2, fi, f2))
            s1 = jnp.where(lt1, d, s1)
            f1 = jnp.where(lt1, fi, f1)
        t1 = t2 = t3 = jnp.full((1, LANES), INF)
        g1 = g2 = g3 = jnp.zeros((1, LANES), jnp.float32)
        ovl = jnp.full((1, LANES), INF)
        for src_s, src_g in ((s1, f1), (s2, f2)):
            for s in range(8):
                v = src_s[s:s + 1, :]
                gg = src_g[s:s + 1, :]
                lt1_ = (v < t1) | ((v == t1) & (gg < g1))
                lt2_ = (v < t2) | ((v == t2) & (gg < g2))
                lt3_ = (v < t3) | ((v == t3) & (gg < g3))
                ovl = jnp.minimum(ovl, jnp.where(lt3_, t3, v))
                t3 = jnp.where(lt2_, t2, jnp.where(lt3_, v, t3))
                g3 = jnp.where(lt2_, g2, jnp.where(lt3_, gg, g3))
                t2 = jnp.where(lt1_, t1, jnp.where(lt2_, v, t2))
                g2 = jnp.where(lt1_, g1, jnp.where(lt2_, gg, g2))
                t1 = jnp.where(lt1_, v, t1)
                g1 = jnp.where(lt1_, gg, g1)
        ov_k = jnp.minimum(ovl, jnp.min(ovp, axis=0, keepdims=True))
        row = sub_i == k
        for arr, val in ((0, t1), (1, t2), (2, t3)):
            T[arr] = jnp.where(row, jnp.broadcast_to(val, (8, LANES)), T[arr])
        for arr, val in ((0, g1), (1, g2), (2, g3)):
            G[arr] = jnp.where(row, jnp.broadcast_to(val, (8, LANES)), G[arr])
        OV = jnp.where(row, jnp.broadcast_to(ov_k, (8, LANES)), OV)

    S1, S2, S3 = T
    H1, H2, H3 = G
    res = jnp.zeros((8, LANES), jnp.float32)
    v16 = None
    for j in range(KNN):
        m = jnp.min(S1, axis=1, keepdims=True)
        v16 = m
        f = jnp.min(jnp.where(S1 == m, H1, BIGF), axis=1, keepdims=True)
        res = jnp.where(lane_i == j, jnp.broadcast_to(f, (8, LANES)), res)
        hit = H1 == f
        S1 = jnp.where(hit, S2, S1)
        H1 = jnp.where(hit, H2, H1)
        S2 = jnp.where(hit, S3, S2)
        H2 = jnp.where(hit, H3, H2)
        S3 = jnp.where(hit, INF, S3)
    nbr_ref[...] = res

    badm = (OV <= v16).astype(jnp.float32)

    @pl.when(jnp.max(badm) > 0.0)
    def _():
        for k in range(CB):
            bk = jnp.max(jnp.where(sub_i == k, badm, 0.0))

            @pl.when(bk > 0.0)
            def _():
                acc = [tile_d(t, k) for t in range(NTIL)]
                fixed = jnp.zeros((8, LANES), jnp.float32)
                for j in range(KNN):
                    m = acc[0]
                    for t in range(1, NTIL):
                        m = jnp.minimum(m, acc[t])
                    mv = jnp.min(m)
                    idx = jnp.float32(BIGI)
                    for t in range(NTIL):
                        idx = jnp.minimum(idx, jnp.min(jnp.where(
                            acc[t] == mv, float(t * 1024) + pos_f, BIGF)))
                    fixed = jnp.where(lane_i == j, idx, fixed)
                    for t in range(NTIL):
                        acc[t] = jnp.where(float(t * 1024) + pos_f == idx,
                                           INF, acc[t])
                nbr_ref[...] = jnp.where(sub_i == k, fixed, nbr_ref[...])


def _knn(px, py, pz, p2, subpos, interpret=False):
    return pl.pallas_call(
        _knn_body,
        grid=(NCLUST // CB,),
        out_shape=jax.ShapeDtypeStruct((NCLUST, LANES), jnp.float32),
        in_specs=[pl.BlockSpec(memory_space=pltpu.VMEM)] * 4
        + [pl.BlockSpec(memory_space=pltpu.SMEM)],
        out_specs=pl.BlockSpec((CB, LANES), lambda b: (b, 0)),
        interpret=interpret,
    )(px, py, pz, p2, subpos)


def _stats_body(x_ref, w_ref, b_ref, sums_ref):
    blk = pl.program_id(0)
    h = jnp.dot(x_ref[...], w_ref[...], preferred_element_type=jnp.float32)
    h = h + b_ref[...]
    s1 = jnp.sum(h, axis=0, keepdims=True)
    s2 = jnp.sum(h * h, axis=0, keepdims=True)
    upd = jnp.concatenate([s1, s2], axis=0)

    @pl.when(blk == 0)
    def _():
        sums_ref[...] = jnp.zeros_like(sums_ref)

    sums_ref[...] += upd


def _stats(x, W, b2, interpret=False):
    return pl.pallas_call(
        _stats_body,
        grid=(NBLK,),
        out_shape=jax.ShapeDtypeStruct((2, COUT), jnp.float32),
        in_specs=[
            pl.BlockSpec((MROWS, CIN), lambda b: (b, 0)),
            pl.BlockSpec(memory_space=pltpu.VMEM),
            pl.BlockSpec(memory_space=pltpu.VMEM),
        ],
        out_specs=pl.BlockSpec((2, COUT), lambda b: (0, 0)),
        interpret=interpret,
    )(x, W, b2)


def _mlp_body(x_ref, w_ref, b_ref, g_ref, beta_ref, sums_ref, h_ref):
    h = jnp.dot(x_ref[...], w_ref[...], preferred_element_type=jnp.float32)
    h = h + b_ref[...]
    n = jnp.float32(NPTS)
    mean = sums_ref[0:1, :] / n
    var = sums_ref[1:2, :] / n - mean * mean
    std = jnp.sqrt(var + 1e-5)
    h = (h - mean) / std * g_ref[...] + beta_ref[...]
    h_ref[...] = jnp.maximum(h, 0.0)


def _mlp(x, W, b2, g2, beta2, sums, interpret=False):
    return pl.pallas_call(
        _mlp_body,
        grid=(NBLK,),
        out_shape=jax.ShapeDtypeStruct((NPTS, COUT), jnp.float32),
        in_specs=[
            pl.BlockSpec((MROWS, CIN), lambda b: (b, 0)),
            pl.BlockSpec(memory_space=pltpu.VMEM),
            pl.BlockSpec(memory_space=pltpu.VMEM),
            pl.BlockSpec(memory_space=pltpu.VMEM),
            pl.BlockSpec(memory_space=pltpu.VMEM),
            pl.BlockSpec(memory_space=pltpu.VMEM),
        ],
        out_specs=pl.BlockSpec((MROWS, COUT), lambda b: (b, 0)),
        interpret=interpret,
    )(x, W, b2, g2, beta2, sums)


NWORK = 32
CPW = NCLUST // NWORK
RPW = CPW * KNN


def _gmax_body(nbr_hbm, h_hbm, idxs_hbm, batch_hbm, out_hbm, sb_hbm,
               idx_v, rows_v, out_v, ci_v, cb_v, sem):
    wid = lax.axis_index("s") * 2 + lax.axis_index("c")
    base = wid * CPW
    pltpu.sync_copy(nbr_hbm.at[pl.ds(base * KNN, RPW)], idx_v)
    pltpu.async_copy(h_hbm.at[idx_v], rows_v, sem).wait()

    def cl(ci, carry):
        for col in range(COUT // 16):
            acc = rows_v[ci * KNN, pl.ds(col * 16, 16)]
            for r in range(1, KNN):
                acc = jnp.maximum(acc, rows_v[ci * KNN + r, pl.ds(col * 16, 16)])
            out_v[ci, pl.ds(col * 16, 16)] = acc
        return carry

    lax.fori_loop(0, CPW, cl, jnp.int32(0))
    pltpu.sync_copy(out_v, out_hbm.at[pl.ds(base, CPW)])

    pltpu.sync_copy(idxs_hbm.at[pl.ds(base, CPW)], ci_v)
    pltpu.async_copy(batch_hbm.at[ci_v], cb_v, sem).wait()
    pltpu.sync_copy(cb_v, sb_hbm.at[pl.ds(base, CPW)])


def _gmax(nbr_flat, h, idxs, batch):
    mesh = plsc.VectorSubcoreMesh(core_axis_name="c", subcore_axis_name="s")
    fn = functools.partial(
        pl.kernel,
        mesh=mesh,
        out_type=[
            jax.ShapeDtypeStruct((NCLUST, COUT), jnp.float32),
            jax.ShapeDtypeStruct((NCLUST,), jnp.int32),
        ],
        scratch_types=[
            pltpu.VMEM((RPW,), jnp.int32),
            pltpu.VMEM((RPW, COUT), jnp.float32),
            pltpu.VMEM((CPW, COUT), jnp.float32),
            pltpu.VMEM((CPW,), jnp.int32),
            pltpu.VMEM((CPW,), jnp.int32),
            pltpu.SemaphoreType.DMA,
        ],
    )(_gmax_body)
    return fn(nbr_flat, h, idxs, batch)


def kernel(x, pos, batch, W, b, gamma, beta):
    posp = jnp.pad(pos, ((0, NPAD - NPTS), (0, 0)))
    px = posp[:, 0].reshape(ROWS, LANES)
    py = posp[:, 1].reshape(ROWS, LANES)
    pz = posp[:, 2].reshape(ROWS, LANES)

    idxs, subpos_flat, p2 = _fps(px, py, pz)
    subpos = subpos_flat.reshape(NCLUST, 3)
    nbr_v = _knn(px.astype(jnp.bfloat16), py.astype(jnp.bfloat16),
                 pz.astype(jnp.bfloat16), p2, subpos_flat)
    nbr = nbr_v[:, :KNN].astype(jnp.int32).reshape(-1)

    b2 = b.reshape(1, COUT)
    g2 = gamma.reshape(1, COUT)
    beta2 = beta.reshape(1, COUT)
    sums = _stats(x, W, b2)
    h = _mlp(x, W, b2, g2, beta2, sums)

    out, sub_batch = _gmax(nbr, h, idxs, batch)
    return (out, subpos, sub_batch)


# speedup vs baseline: 5.7787x; 1.0985x over previous
import functools

import jax
import jax.numpy as jnp
from jax import lax
from jax.experimental import pallas as pl
from jax.experimental.pallas import tpu as pltpu
from jax.experimental.pallas import tpu_sc as plsc

NPTS = 50000
LANES = 128
ROWS = 392
NPAD = ROWS * LANES
NCLUST = 1024
KNN = 16
CIN = 128
COUT = 128
MROWS = 2000
NBLK = NPTS // MROWS


def _iota2(shape):
    r = lax.broadcasted_iota(jnp.int32, shape, 0)
    l = lax.broadcasted_iota(jnp.int32, shape, 1)
    return r * LANES + l


def _fps_body(px_ref, py_ref, pz_ref, idxs_ref, subpos_ref, p2_ref, d_ref):
    px = px_ref[...]
    py = py_ref[...]
    pz = pz_ref[...]
    flat = _iota2((ROWS, LANES))
    lane1 = lax.broadcasted_iota(jnp.int32, (1, LANES), 1)
    valid = flat < NPTS
    pos_f = (lax.broadcasted_iota(jnp.int32, (8, LANES), 0) * LANES
             + lax.broadcasted_iota(jnp.int32, (8, LANES), 1)).astype(jnp.float32)
    BIGF = jnp.float32(3e38)

    x0 = px[0:1, 0:1]
    y0 = py[0:1, 0:1]
    z0 = pz[0:1, 0:1]
    dx = px - x0
    dy = py - y0
    dz = pz - z0
    d = (dx * dx + dy * dy) + dz * dz
    d = jnp.where(valid, d, -1.0)
    d_ref[...] = d

    p2_ref[...] = (px * px + py * py) + pz * pz

    idxs_ref[0] = jnp.int32(0)
    subpos_ref[0] = x0[0, 0]
    subpos_ref[1] = y0[0, 0]
    subpos_ref[2] = z0[0, 0]

    rm = jnp.full((8, LANES), -INF)
    am = jnp.zeros((8, LANES), jnp.float32)
    for t in range(NTIL):
        dt = d[8 * t:8 * t + 8, :]
        gt = dt > rm
        am = jnp.where(gt, float(t), am)
        rm = jnp.maximum(rm, dt)

    def body(i, carry):
        rm, am = carry
        m = jnp.max(rm)
        fidx = jnp.min(jnp.where(rm == m, am * 1024.0 + pos_f, BIGF))
        nxt = fidx.astype(jnp.int32)
        idxs_ref[i] = nxt
        r = nxt // LANES
        l = nxt % LANES
        lm = lane1 == l
        rx = px_ref[pl.ds(r, 1), :]
        ry = py_ref[pl.ds(r, 1), :]
        rz = pz_ref[pl.ds(r, 1), :]
        nx = jnp.sum(jnp.where(lm, rx, 0.0))
        ny = jnp.sum(jnp.where(lm, ry, 0.0))
        nz = jnp.sum(jnp.where(lm, rz, 0.0))
        subpos_ref[3 * i] = nx
        subpos_ref[3 * i + 1] = ny
        subpos_ref[3 * i + 2] = nz
        rm2 = jnp.full((8, LANES), -INF)
        am2 = jnp.zeros((8, LANES), jnp.float32)
        for t in range(NTIL):
            sl = pl.ds(8 * t, 8)
            ddx = px_ref[sl, :] - nx
            ddy = py_ref[sl, :] - ny
            ddz = pz_ref[sl, :] - nz
            dd = (ddx * ddx + ddy * ddy) + ddz * ddz
            dn = jnp.minimum(d_ref[sl, :], dd)
            d_ref[sl, :] = dn
            gt = dn > rm2
            am2 = jnp.where(gt, float(t), am2)
            rm2 = jnp.maximum(rm2, dn)
        return (rm2, am2)

    lax.fori_loop(1, NCLUST, body, (rm, am))


def _fps(px, py, pz, interpret=False):
    return pl.pallas_call(
        _fps_body,
        out_shape=[
            jax.ShapeDtypeStruct((NCLUST,), jnp.int32),
            jax.ShapeDtypeStruct((NCLUST * 3,), jnp.float32),
            jax.ShapeDtypeStruct((ROWS, LANES), jnp.float32),
        ],
        in_specs=[pl.BlockSpec(memory_space=pltpu.VMEM)] * 3,
        out_specs=[
            pl.BlockSpec(memory_space=pltpu.SMEM),
            pl.BlockSpec(memory_space=pltpu.SMEM),
            pl.BlockSpec(memory_space=pltpu.VMEM),
        ],
        scratch_shapes=[pltpu.VMEM((ROWS, LANES), jnp.float32)],
        interpret=interpret,
    )(px, py, pz)


CB = 8
NTIL = ROWS // 8
BIGI = 2 ** 30
INF = float("inf")


def _knn_body(px_ref, py_ref, pz_ref, p2_ref, subpos_ref, nbr_ref):
    g = pl.program_id(0)
    sub_i = lax.broadcasted_iota(jnp.int32, (8, LANES), 0)
    lane_i = lax.broadcasted_iota(jnp.int32, (8, LANES), 1)
    pos_idx = sub_i * LANES + lane_i
    pos_f = pos_idx.astype(jnp.float32)
    last_valid = (NTIL - 1) * 1024 + pos_idx < NPTS

    coef = []
    for k in range(CB):
        c = g * CB + k
        cx = subpos_ref[3 * c]
        cy = subpos_ref[3 * c + 1]
        cz = subpos_ref[3 * c + 2]
        c2 = (cx * cx + cy * cy) + cz * cz
        coef.append((cx.astype(jnp.bfloat16).astype(jnp.float32),
                     cy.astype(jnp.bfloat16).astype(jnp.float32),
                     cz.astype(jnp.bfloat16).astype(jnp.float32), c2))

    def tile_d(t, k):
        r = pl.ds(8 * t, 8)
        pxb = px_ref[r, :].astype(jnp.float32)
        pyb = py_ref[r, :].astype(jnp.float32)
        pzb = pz_ref[r, :].astype(jnp.float32)
        cxb, cyb, czb, c2 = coef[k]
        dot = (cxb * pxb + cyb * pyb) + czb * pzb
        d = (c2 + p2_ref[r, :]) - 2.0 * dot
        if t == NTIL - 1:
            d = jnp.where(last_valid, d, INF)
        return d

    BIGF = jnp.float32(3e38)

    T = [jnp.full((8, LANES), INF) for _ in range(3)]
    G = [jnp.zeros((8, LANES), jnp.float32) for _ in range(3)]
    OV = jnp.full((8, LANES), INF)

    for k in range(CB):
        s1 = jnp.full((8, LANES), INF)
        s2 = jnp.full((8, LANES), INF)
        f1 = jnp.zeros((8, LANES), jnp.float32)
        f2 = jnp.zeros((8, LANES), jnp.float32)
        ovp = jnp.full((8, LANES), INF)
        for t in range(NTIL):
            fi = float(t * 1024) + pos_f
            d = tile_d(t, k)
            lt1 = d < s1
            lt2 = d < s2
            ovp = jnp.minimum(ovp, jnp.where(lt2, s2, d))
            s2 = jnp.where(lt1, s1, jnp.where(lt2, d, s2))
            f2 = jnp.where(lt1, f1, jnp.where(lt2, fi, f2))
            s1 = jnp.where(lt1, d, s1)
            f1 = jnp.where(lt1, fi, f1)
        t1 = t2 = t3 = jnp.full((1, LANES), INF)
        g1 = g2 = g3 = jnp.zeros((1, LANES), jnp.float32)
        ovl = jnp.full((1, LANES), INF)
        for src_s, src_g in ((s1, f1), (s2, f2)):
            for s in range(8):
                v = src_s[s:s + 1, :]
                gg = src_g[s:s + 1, :]
                lt1_ = (v < t1) | ((v == t1) & (gg < g1))
                lt2_ = (v < t2) | ((v == t2) & (gg < g2))
                lt3_ = (v < t3) | ((v == t3) & (gg < g3))
                ovl = jnp.minimum(ovl, jnp.where(lt3_, t3, v))
                t3 = jnp.where(lt2_, t2, jnp.where(lt3_, v, t3))
                g3 = jnp.where(lt2_, g2, jnp.where(lt3_, gg, g3))
                t2 = jnp.where(lt1_, t1, jnp.where(lt2_, v, t2))
                g2 = jnp.where(lt1_, g1, jnp.where(lt2_, gg, g2))
                t1 = jnp.where(lt1_, v, t1)
                g1 = jnp.where(lt1_, gg, g1)
        ov_k = jnp.minimum(ovl, jnp.min(ovp, axis=0, keepdims=True))
        row = sub_i == k
        for arr, val in ((0, t1), (1, t2), (2, t3)):
            T[arr] = jnp.where(row, jnp.broadcast_to(val, (8, LANES)), T[arr])
        for arr, val in ((0, g1), (1, g2), (2, g3)):
            G[arr] = jnp.where(row, jnp.broadcast_to(val, (8, LANES)), G[arr])
        OV = jnp.where(row, jnp.broadcast_to(ov_k, (8, LANES)), OV)

    S1, S2, S3 = T
    H1, H2, H3 = G
    res = jnp.zeros((8, LANES), jnp.float32)
    v16 = None
    for j in range(KNN):
        m = jnp.min(S1, axis=1, keepdims=True)
        v16 = m
        f = jnp.min(jnp.where(S1 == m, H1, BIGF), axis=1, keepdims=True)
        res = jnp.where(lane_i == j, jnp.broadcast_to(f, (8, LANES)), res)
        hit = H1 == f
        S1 = jnp.where(hit, S2, S1)
        H1 = jnp.where(hit, H2, H1)
        S2 = jnp.where(hit, S3, S2)
        H2 = jnp.where(hit, H3, H2)
        S3 = jnp.where(hit, INF, S3)
    nbr_ref[...] = res

    badm = (OV <= v16).astype(jnp.float32)

    @pl.when(jnp.max(badm) > 0.0)
    def _():
        for k in range(CB):
            bk = jnp.max(jnp.where(sub_i == k, badm, 0.0))

            @pl.when(bk > 0.0)
            def _():
                acc = [tile_d(t, k) for t in range(NTIL)]
                fixed = jnp.zeros((8, LANES), jnp.float32)
                for j in range(KNN):
                    m = acc[0]
                    for t in range(1, NTIL):
                        m = jnp.minimum(m, acc[t])
                    mv = jnp.min(m)
                    idx = jnp.float32(BIGI)
                    for t in range(NTIL):
                        idx = jnp.minimum(idx, jnp.min(jnp.where(
                            acc[t] == mv, float(t * 1024) + pos_f, BIGF)))
                    fixed = jnp.where(lane_i == j, idx, fixed)
                    for t in range(NTIL):
                        acc[t] = jnp.where(float(t * 1024) + pos_f == idx,
                                           INF, acc[t])
                nbr_ref[...] = jnp.where(sub_i == k, fixed, nbr_ref[...])


def _knn(px, py, pz, p2, subpos, interpret=False):
    return pl.pallas_call(
        _knn_body,
        grid=(NCLUST // CB,),
        out_shape=jax.ShapeDtypeStruct((NCLUST, LANES), jnp.float32),
        in_specs=[pl.BlockSpec(memory_space=pltpu.VMEM)] * 4
        + [pl.BlockSpec(memory_space=pltpu.SMEM)],
        out_specs=pl.BlockSpec((CB, LANES), lambda b: (b, 0)),
        interpret=interpret,
    )(px, py, pz, p2, subpos)


def _stats_body(x_ref, w_ref, b_ref, sums_ref):
    blk = pl.program_id(0)
    h = jnp.dot(x_ref[...], w_ref[...], preferred_element_type=jnp.float32)
    h = h + b_ref[...]
    s1 = jnp.sum(h, axis=0, keepdims=True)
    s2 = jnp.sum(h * h, axis=0, keepdims=True)
    upd = jnp.concatenate([s1, s2], axis=0)

    @pl.when(blk == 0)
    def _():
        sums_ref[...] = jnp.zeros_like(sums_ref)

    sums_ref[...] += upd


def _stats(x, W, b2, interpret=False):
    return pl.pallas_call(
        _stats_body,
        grid=(NBLK,),
        out_shape=jax.ShapeDtypeStruct((2, COUT), jnp.float32),
        in_specs=[
            pl.BlockSpec((MROWS, CIN), lambda b: (b, 0)),
            pl.BlockSpec(memory_space=pltpu.VMEM),
            pl.BlockSpec(memory_space=pltpu.VMEM),
        ],
        out_specs=pl.BlockSpec((2, COUT), lambda b: (0, 0)),
        interpret=interpret,
    )(x, W, b2)


def _mlp_body(x_ref, w_ref, b_ref, g_ref, beta_ref, sums_ref, h_ref):
    h = jnp.dot(x_ref[...], w_ref[...], preferred_element_type=jnp.float32)
    h = h + b_ref[...]
    n = jnp.float32(NPTS)
    mean = sums_ref[0:1, :] / n
    var = sums_ref[1:2, :] / n - mean * mean
    std = jnp.sqrt(var + 1e-5)
    h = (h - mean) / std * g_ref[...] + beta_ref[...]
    h_ref[...] = jnp.maximum(h, 0.0)


def _mlp(x, W, b2, g2, beta2, sums, interpret=False):
    return pl.pallas_call(
        _mlp_body,
        grid=(NBLK,),
        out_shape=jax.ShapeDtypeStruct((NPTS, COUT), jnp.float32),
        in_specs=[
            pl.BlockSpec((MROWS, CIN), lambda b: (b, 0)),
            pl.BlockSpec(memory_space=pltpu.VMEM),
            pl.BlockSpec(memory_space=pltpu.VMEM),
            pl.BlockSpec(memory_space=pltpu.VMEM),
            pl.BlockSpec(memory_space=pltpu.VMEM),
            pl.BlockSpec(memory_space=pltpu.VMEM),
        ],
        out_specs=pl.BlockSpec((MROWS, COUT), lambda b: (b, 0)),
        interpret=interpret,
    )(x, W, b2, g2, beta2, sums)


NWORK = 32
CPW = NCLUST // NWORK
RPW = CPW * KNN


def _gmax_body(nbr_hbm, h_hbm, idxs_hbm, batch_hbm, out_hbm, sb_hbm,
               idx_v, rows_v, out_v, ci_v, cb_v, sem):
    wid = lax.axis_index("s") * 2 + lax.axis_index("c")
    base = wid * CPW
    pltpu.sync_copy(nbr_hbm.at[pl.ds(base * KNN, RPW)], idx_v)
    pltpu.async_copy(h_hbm.at[idx_v], rows_v, sem).wait()

    def cl(ci, carry):
        for col in range(COUT // 16):
            acc = rows_v[ci * KNN, pl.ds(col * 16, 16)]
            for r in range(1, KNN):
                acc = jnp.maximum(acc, rows_v[ci * KNN + r, pl.ds(col * 16, 16)])
            out_v[ci, pl.ds(col * 16, 16)] = acc
        return carry

    lax.fori_loop(0, CPW, cl, jnp.int32(0))
    pltpu.sync_copy(out_v, out_hbm.at[pl.ds(base, CPW)])

    pltpu.sync_copy(idxs_hbm.at[pl.ds(base, CPW)], ci_v)
    pltpu.async_copy(batch_hbm.at[ci_v], cb_v, sem).wait()
    pltpu.sync_copy(cb_v, sb_hbm.at[pl.ds(base, CPW)])


def _gmax(nbr_flat, h, idxs, batch):
    mesh = plsc.VectorSubcoreMesh(core_axis_name="c", subcore_axis_name="s")
    fn = functools.partial(
        pl.kernel,
        mesh=mesh,
        out_type=[
            jax.ShapeDtypeStruct((NCLUST, COUT), jnp.float32),
            jax.ShapeDtypeStruct((NCLUST,), jnp.int32),
        ],
        scratch_types=[
            pltpu.VMEM((RPW,), jnp.int32),
            pltpu.VMEM((RPW, COUT), jnp.float32),
            pltpu.VMEM((CPW, COUT), jnp.float32),
            pltpu.VMEM((CPW,), jnp.int32),
            pltpu.VMEM((CPW,), jnp.int32),
            pltpu.SemaphoreType.DMA,
        ],
    )(_gmax_body)
    return fn(nbr_flat, h, idxs, batch)


def kernel(x, pos, batch, W, b, gamma, beta):
    posp = jnp.pad(pos, ((0, NPAD - NPTS), (0, 0)))
    px = posp[:, 0].reshape(ROWS, LANES)
    py = posp[:, 1].reshape(ROWS, LANES)
    pz = posp[:, 2].reshape(ROWS, LANES)

    idxs, subpos_flat, p2 = _fps(px, py, pz)
    subpos = subpos_flat.reshape(NCLUST, 3)
    nbr_v = _knn(px.astype(jnp.bfloat16), py.astype(jnp.bfloat16),
                 pz.astype(jnp.bfloat16), p2, subpos_flat)
    nbr = nbr_v[:, :KNN].astype(jnp.int32).reshape(-1)

    b2 = b.reshape(1, COUT)
    g2 = gamma.reshape(1, COUT)
    beta2 = beta.reshape(1, COUT)
    sums = _stats(x, W, b2)
    h = _mlp(x, W, b2, g2, beta2, sums)

    out, sub_batch = _gmax(nbr, h, idxs, batch)
    return (out, subpos, sub_batch)
